# baseline (speedup 1.0000x reference)
import functools

import jax
import jax.numpy as jnp
from jax import lax
from jax.experimental import pallas as pl
from jax.experimental.pallas import tpu as pltpu
from jax.experimental.pallas import tpu_sc as plsc

GROUPS = 512
KNEIGHBORS = 32
CHANNEL = 256
NPTS = 4096
NBATCH = 8
NDIM = 5
CTOT = CHANNEL + NDIM
OUTW = CTOT + CHANNEL
PW = 272
NC, NS = 2, 16
NW = NC * NS
GPT = (NBATCH * GROUPS) // NW
TPB = NW // NBATCH


def _index_points(points, idx):
    B = points.shape[0]
    if idx.ndim == 2:
        return points[jnp.arange(B)[:, None], idx]
    else:
        return points[jnp.arange(B)[:, None, None], idx]


def _fps_kernel(xyzt_ref, far0_ref, cent_ref, cen_ref):
    B, N = NBATCH, NPTS
    S = GROUPS
    x = [xyzt_ref[c] for c in range(NDIM)]
    lane = jax.lax.broadcasted_iota(jnp.int32, (B, N), 1)
    col = jax.lax.broadcasted_iota(jnp.int32, (B, S), 1)
    dist0 = jnp.full((B, N), 1e10, dtype=jnp.float32)
    far0 = far0_ref[...]
    cent0 = jnp.zeros((B, S), jnp.int32)
    cen0 = tuple(jnp.zeros((B, S), jnp.float32) for _ in range(NDIM))

    def body(i, carry):
        dist, far, cent = carry[0], carry[1], carry[2]
        cacc = list(carry[3:])
        cent = jnp.where(col == i, jnp.broadcast_to(far, (B, S)), cent)
        mask = lane == far
        cenc = [jnp.sum(jnp.where(mask, xc, 0.0), axis=1, keepdims=True)
                for xc in x]
        cacc = [jnp.where(col == i, jnp.broadcast_to(cc, (B, S)), ca)
                for cc, ca in zip(cenc, cacc)]
        d = (x[0] - cenc[0]) ** 2
        for c in range(1, NDIM):
            d = d + (x[c] - cenc[c]) ** 2
        dist = jnp.minimum(dist, d)
        maxv = jnp.max(dist, axis=1, keepdims=True)
        far = jnp.min(jnp.where(dist == maxv, lane, N), axis=1, keepdims=True)
        return (dist, far, cent, *cacc)

    out = jax.lax.fori_loop(0, S, body, (dist0, far0, cent0, *cen0))
    cent_ref[...] = out[2]
    for c in range(NDIM):
        cen_ref[c] = out[3 + c]


def _fps_pallas(xyz):
    xyzt = jnp.transpose(xyz, (2, 0, 1))
    far0 = jax.random.randint(jax.random.key(1), (NBATCH, 1), 0, NPTS,
                              dtype=jnp.int32)
    cent, cen = pl.pallas_call(
        _fps_kernel,
        out_shape=(
            jax.ShapeDtypeStruct((NBATCH, GROUPS), jnp.int32),
            jax.ShapeDtypeStruct((NDIM, NBATCH, GROUPS), jnp.float32),
        ),
    )(xyzt, far0)
    new_xyz = jnp.transpose(cen, (1, 2, 0))
    return cent, new_xyz


def _topk_kernel(dist_ref, idx_ref):
    S, N, K = GROUPS, NPTS, KNEIGHBORS
    lane = lax.broadcasted_iota(jnp.int32, (S, N), 1)
    colk = lax.broadcasted_iota(jnp.int32, (S, K), 1)
    negd = -dist_ref[0]
    acc0 = jnp.zeros((S, K), jnp.int32)

    def body(k, carry):
        negd, acc = carry
        maxv = jnp.max(negd, axis=1, keepdims=True)
        idx = jnp.min(jnp.where(negd == maxv, lane, N), axis=1, keepdims=True)
        acc = jnp.where(colk == k, jnp.broadcast_to(idx, (S, K)), acc)
        negd = jnp.where(lane == idx, -jnp.inf, negd)
        return (negd, acc)

    _, acc = lax.fori_loop(0, K, body, (negd, acc0))
    idx_ref[0] = acc


def _topk_pallas(dist):
    return pl.pallas_call(
        _topk_kernel,
        grid=(NBATCH,),
        in_specs=[pl.BlockSpec((1, GROUPS, NPTS), lambda b: (b, 0, 0))],
        out_specs=pl.BlockSpec((1, GROUPS, KNEIGHBORS), lambda b: (b, 0, 0)),
        out_shape=jax.ShapeDtypeStruct((NBATCH, GROUPS, KNEIGHBORS),
                                       jnp.int32),
        compiler_params=pltpu.CompilerParams(
            dimension_semantics=("parallel",)),
    )(dist)


_MESH = functools.partial(
    plsc.VectorSubcoreMesh, core_axis_name="c", subcore_axis_name="s")


def _iota16():
    return lax.broadcasted_iota(jnp.int32, (16,), 0)


def _splat_i32(v):
    return jnp.full((16,), v, dtype=jnp.int32)


def _sc_stats_body(pointsf, xyzt_hbm, gidx_hbm, sums_hbm, part_hbm,
                   idxbuf, xyzv, sumsb, pbuf, pvec, sem):
    w = lax.axis_index("s") * NC + lax.axis_index("c")
    b = w // TPB
    g0 = (w % TPB) * GPT
    bN = b * NPTS
    pltpu.sync_copy(gidx_hbm.at[b, pl.ds(g0, GPT)], idxbuf)
    pltpu.sync_copy(xyzt_hbm.at[b], xyzv)
    inv32 = jnp.float32(1.0 / KNEIGHBORS)

    def body(g, carry):
        gsq_t, m2_t = carry
        pltpu.async_copy(pointsf.at[idxbuf.at[g]], pbuf, sem).wait()
        gsq = jnp.zeros((16,), jnp.float32)
        m2 = jnp.zeros((16,), jnp.float32)
        for j in range(16):
            acc = pbuf[0, pl.ds(16 * j, 16)]
            gsq = gsq + acc * acc
            for k in range(1, KNEIGHBORS):
                v = pbuf[k, pl.ds(16 * j, 16)]
                acc = acc + v
                gsq = gsq + v * v
            sumsb[pl.ds(g * PW + 16 * j, 16)] = acc
            m = acc * inv32
            m2 = m2 + m * m
        n0 = idxbuf[g, pl.ds(0, 16)] - bN
        n1 = idxbuf[g, pl.ds(16, 16)] - bN
        li = _iota16()
        xs = jnp.zeros((16,), jnp.float32)
        for c in range(NDIM):
            x0 = plsc.load_gather(xyzv, [n0 + c * NPTS])
            x1 = plsc.load_gather(xyzv, [n1 + c * NPTS])
            gsq = gsq + x0 * x0
            gsq = gsq + x1 * x1
            sc = jnp.sum(x0 + x1)
            xs = jnp.where(li == c, sc, xs)
        sumsb[pl.ds(g * PW + 256, 16)] = xs
        mx = xs * inv32
        m2 = m2 + mx * mx
        return (gsq_t + gsq, m2_t + m2)

    gsq_t, m2_t = lax.fori_loop(0, GPT, body,
                                (jnp.zeros((16,), jnp.float32),
                                 jnp.zeros((16,), jnp.float32)))
    pltpu.sync_copy(sumsb, sums_hbm.at[b, pl.ds(g0 * PW, GPT * PW)])
    s1 = jnp.sum(gsq_t)
    s2 = jnp.sum(m2_t)
    li = _iota16()
    pv = jnp.where(li == 0, s1, jnp.where(li == 1, s2, jnp.float32(0.0)))
    pvec[...] = pv
    pltpu.sync_copy(pvec, part_hbm.at[w])


def _sc_stats(pointsf, xyzt, gidx):
    k = pl.kernel(
        _sc_stats_body,
        out_type=(
            jax.ShapeDtypeStruct((NBATCH, GROUPS * PW), jnp.float32),
            jax.ShapeDtypeStruct((NW, 16), jnp.float32),
        ),
        mesh=_MESH(),
        compiler_params=pltpu.CompilerParams(use_tc_tiling_on_sc=False, needs_layout_passes=False),
        scratch_types=[
            pltpu.VMEM((GPT, KNEIGHBORS + 1), jnp.int32),
            pltpu.VMEM((NDIM * NPTS,), jnp.float32),
            pltpu.VMEM((GPT * PW,), jnp.float32),
            pltpu.VMEM((KNEIGHBORS + 1, CHANNEL), jnp.float32),
            pltpu.VMEM((16,), jnp.float32),
            pltpu.SemaphoreType.DMA,
        ],
    )
    return k(pointsf, xyzt, gidx)


def _sc_write_body(pointsf, xyzt_hbm, gidx_hbm, sums_hbm, a_hbm, b_hbm,
                   out_hbm,
                   idxbuf, xyzv, sumsb, avec, bvec, pbuf, asm, sem):
    w = lax.axis_index("s") * NC + lax.axis_index("c")
    b = w // TPB
    g0 = (w % TPB) * GPT
    bN = b * NPTS
    pltpu.sync_copy(gidx_hbm.at[b, pl.ds(g0, GPT)], idxbuf)
    pltpu.sync_copy(xyzt_hbm.at[b], xyzv)
    pltpu.sync_copy(sums_hbm.at[b, pl.ds(g0 * PW, GPT * PW)], sumsb)
    pltpu.sync_copy(a_hbm.at[b], avec)
    pltpu.sync_copy(b_hbm.at[b], bvec)
    inv32 = jnp.float32(1.0 / KNEIGHBORS)
    li = _iota16()
    ac = [plsc.load_gather(avec, [_splat_i32(256 + c)]) for c in range(NDIM)]
    bc = [plsc.load_gather(bvec, [_splat_i32(256 + c)]) for c in range(NDIM)]
    off0 = [li * OUTW + (CHANNEL + c) for c in range(NDIM)]
    off1 = [o + 16 * OUTW for o in off0]

    def body(g, carry):
        pltpu.async_copy(pointsf.at[idxbuf.at[g]], pbuf, sem).wait()
        for j in range(16):
            a = avec[pl.ds(16 * j, 16)]
            bb = bvec[pl.ds(16 * j, 16)]
            srow = sumsb[pl.ds(g * PW + 16 * j, 16)]
            sh = bb - (srow * inv32) * a
            for k in range(KNEIGHBORS):
                v = pbuf[k, pl.ds(16 * j, 16)]
                asm[pl.ds(k * OUTW + 16 * j, 16)] = v * a + sh
            rv = pbuf[KNEIGHBORS, pl.ds(16 * j, 16)]
            for k in range(KNEIGHBORS):
                asm[pl.ds(k * OUTW + CTOT + 16 * j, 16)] = rv
        n0 = idxbuf[g, pl.ds(0, 16)] - bN
        n1 = idxbuf[g, pl.ds(16, 16)] - bN
        for c in range(NDIM):
            x0 = plsc.load_gather(xyzv, [n0 + c * NPTS])
            x1 = plsc.load_gather(xyzv, [n1 + c * NPTS])
            mc = plsc.load_gather(sumsb, [_splat_i32(0) + (g * PW + 256 + c)])
            sh = bc[c] - (mc * inv32) * ac[c]
            plsc.store_scatter(asm, [off0[c]], x0 * ac[c] + sh)
            plsc.store_scatter(asm, [off1[c]], x1 * ac[c] + sh)
        pltpu.sync_copy(asm, out_hbm.at[b, g0 + g])
        return carry

    lax.fori_loop(0, GPT, body, 0)


def _sc_write(pointsf, xyzt, gidx, sums, avals, bvals):
    k = pl.kernel(
        _sc_write_body,
        out_type=jax.ShapeDtypeStruct(
            (NBATCH, GROUPS, KNEIGHBORS * OUTW), jnp.float32),
        mesh=_MESH(),
        compiler_params=pltpu.CompilerParams(use_tc_tiling_on_sc=False, needs_layout_passes=False),
        scratch_types=[
            pltpu.VMEM((GPT, KNEIGHBORS + 1), jnp.int32),
            pltpu.VMEM((NDIM * NPTS,), jnp.float32),
            pltpu.VMEM((GPT * PW,), jnp.float32),
            pltpu.VMEM((PW,), jnp.float32),
            pltpu.VMEM((PW,), jnp.float32),
            pltpu.VMEM((KNEIGHBORS + 1, CHANNEL), jnp.float32),
            pltpu.VMEM((KNEIGHBORS * OUTW,), jnp.float32),
            pltpu.SemaphoreType.DMA,
        ],
    )
    return k(pointsf, xyzt, gidx, sums, avals, bvals)


def kernel(xyz, points, affine_alpha, affine_beta):
    B, N, C = xyz.shape
    S = GROUPS
    K = KNEIGHBORS
    fps_idx, new_xyz = _fps_pallas(xyz)
    dist = -2.0 * jnp.matmul(new_xyz, jnp.transpose(xyz, (0, 2, 1)))
    dist = dist + jnp.sum(new_xyz ** 2, axis=-1)[:, :, None]
    dist = dist + jnp.sum(xyz ** 2, axis=-1)[:, None, :]
    idx = _topk_pallas(dist)

    boff = (jnp.arange(B, dtype=jnp.int32) * N)
    gidx = jnp.concatenate(
        [idx + boff[:, None, None], (fps_idx + boff[:, None])[:, :, None]],
        axis=2).astype(jnp.int32)
    pointsf = points.reshape(B * N, CHANNEL)
    xyzt = jnp.transpose(xyz, (0, 2, 1)).reshape(B, NDIM * N)

    sums, parts = _sc_stats(pointsf, xyzt, gidx)
    pr = parts.reshape(NBATCH, TPB, 16).sum(axis=1)
    ss = pr[:, 0] - K * pr[:, 1]
    M = S * K * CTOT
    std = jnp.sqrt(ss / (M - 1))
    inv = 1.0 / (std + 1e-05)
    af = affine_alpha.reshape(CTOT)
    bf = affine_beta.reshape(CTOT)
    avals = jnp.zeros((NBATCH, PW), jnp.float32).at[:, :CTOT].set(
        inv[:, None] * af[None, :])
    bvals = jnp.zeros((NBATCH, PW), jnp.float32).at[:, :CTOT].set(
        jnp.broadcast_to(bf[None, :], (NBATCH, CTOT)))

    out = _sc_write(pointsf, xyzt, gidx, sums, avals, bvals)
    new_points_out = out.reshape(B, S, K, OUTW)
    return (new_xyz, new_points_out)


# speedup vs baseline: 1.3093x; 1.3093x over previous
import functools

import jax
import jax.numpy as jnp
from jax import lax
from jax.experimental import pallas as pl
from jax.experimental.pallas import tpu as pltpu
from jax.experimental.pallas import tpu_sc as plsc

GROUPS = 512
KNEIGHBORS = 32
CHANNEL = 256
NPTS = 4096
NBATCH = 8
NDIM = 5
CTOT = CHANNEL + NDIM
OUTW = CTOT + CHANNEL
PW = 272
NC, NS = 2, 16
NW = NC * NS
GPT = (NBATCH * GROUPS) // NW
TPB = NW // NBATCH


def _index_points(points, idx):
    B = points.shape[0]
    if idx.ndim == 2:
        return points[jnp.arange(B)[:, None], idx]
    else:
        return points[jnp.arange(B)[:, None, None], idx]


def _fps_kernel(xyzt_ref, far0_ref, cent_ref, cen_ref):
    B, N = NBATCH, NPTS
    S = GROUPS
    x = [xyzt_ref[c] for c in range(NDIM)]
    lane = jax.lax.broadcasted_iota(jnp.int32, (B, N), 1)
    col = jax.lax.broadcasted_iota(jnp.int32, (B, S), 1)
    dist0 = jnp.full((B, N), 1e10, dtype=jnp.float32)
    far0 = far0_ref[...]
    cent0 = jnp.zeros((B, S), jnp.int32)
    cen0 = tuple(jnp.zeros((B, S), jnp.float32) for _ in range(NDIM))

    def body(i, carry):
        dist, far, cent = carry[0], carry[1], carry[2]
        cacc = list(carry[3:])
        cent = jnp.where(col == i, jnp.broadcast_to(far, (B, S)), cent)
        mask = lane == far
        cenc = [jnp.sum(jnp.where(mask, xc, 0.0), axis=1, keepdims=True)
                for xc in x]
        cacc = [jnp.where(col == i, jnp.broadcast_to(cc, (B, S)), ca)
                for cc, ca in zip(cenc, cacc)]
        d = (x[0] - cenc[0]) ** 2
        for c in range(1, NDIM):
            d = d + (x[c] - cenc[c]) ** 2
        dist = jnp.minimum(dist, d)
        maxv = jnp.max(dist, axis=1, keepdims=True)
        far = jnp.min(jnp.where(dist == maxv, lane, N), axis=1, keepdims=True)
        return (dist, far, cent, *cacc)

    out = jax.lax.fori_loop(0, S, body, (dist0, far0, cent0, *cen0))
    cent_ref[...] = out[2]
    for c in range(NDIM):
        cen_ref[c] = out[3 + c]


def _fps_pallas(xyz):
    xyzt = jnp.transpose(xyz, (2, 0, 1))
    far0 = jax.random.randint(jax.random.key(1), (NBATCH, 1), 0, NPTS,
                              dtype=jnp.int32)
    cent, cen = pl.pallas_call(
        _fps_kernel,
        out_shape=(
            jax.ShapeDtypeStruct((NBATCH, GROUPS), jnp.int32),
            jax.ShapeDtypeStruct((NDIM, NBATCH, GROUPS), jnp.float32),
        ),
    )(xyzt, far0)
    new_xyz = jnp.transpose(cen, (1, 2, 0))
    return cent, new_xyz


def _topk_kernel(dist_ref, idx_ref, negd_ref):
    S, N, K = GROUPS, NPTS, KNEIGHBORS
    CH, CW = 32, 128
    lane128 = lax.broadcasted_iota(jnp.int32, (S, CW), 1)
    colk = lax.broadcasted_iota(jnp.int32, (S, K), 1)
    negd_ref[...] = -dist_ref[0]
    acc0 = jnp.zeros((S, K), jnp.int32)
    idxp0 = jnp.full((S, 1), -1, jnp.int32)
    NEG = jnp.float32(-jnp.inf)

    def body(k, carry):
        idxp, acc = carry
        m_lane = None
        for c in range(CH):
            v = negd_ref[:, c * CW:(c + 1) * CW]
            v = jnp.where(lane128 == idxp - c * CW, NEG, v)
            negd_ref[:, c * CW:(c + 1) * CW] = v
            m_lane = v if m_lane is None else jnp.maximum(m_lane, v)
        cacc = jnp.full((S, CW), CH, jnp.int32)
        for c in range(CH - 1, -1, -1):
            v = negd_ref[:, c * CW:(c + 1) * CW]
            cacc = jnp.where(v == m_lane, c, cacc)
        maxv = jnp.max(m_lane, axis=1, keepdims=True)
        gcand = cacc * CW + lane128
        idx = jnp.min(jnp.where(m_lane == maxv, gcand, N), axis=1,
                      keepdims=True)
        acc = jnp.where(colk == k, jnp.broadcast_to(idx, (S, K)), acc)
        return (idx, acc)

    _, acc = lax.fori_loop(0, K, body, (idxp0, acc0))
    idx_ref[0] = acc


def _topk_pallas(dist):
    return pl.pallas_call(
        _topk_kernel,
        grid=(NBATCH,),
        in_specs=[pl.BlockSpec((1, GROUPS, NPTS), lambda b: (b, 0, 0))],
        out_specs=pl.BlockSpec((1, GROUPS, KNEIGHBORS), lambda b: (b, 0, 0)),
        out_shape=jax.ShapeDtypeStruct((NBATCH, GROUPS, KNEIGHBORS),
                                       jnp.int32),
        scratch_shapes=[pltpu.VMEM((GROUPS, NPTS), jnp.float32)],
        compiler_params=pltpu.CompilerParams(
            dimension_semantics=("arbitrary",)),
    )(dist)


_MESH = functools.partial(
    plsc.VectorSubcoreMesh, core_axis_name="c", subcore_axis_name="s")


def _iota16():
    return lax.broadcasted_iota(jnp.int32, (16,), 0)


def _splat_i32(v):
    return jnp.full((16,), v, dtype=jnp.int32)


def _sc_stats_body(pointsf, xyzt_hbm, gidx_hbm, sums_hbm, part_hbm,
                   idxbuf, xyzv, sumsb, pbuf, pvec, sem):
    w = lax.axis_index("s") * NC + lax.axis_index("c")
    b = w // TPB
    g0 = (w % TPB) * GPT
    bN = b * NPTS
    pltpu.sync_copy(gidx_hbm.at[b, pl.ds(g0, GPT)], idxbuf)
    pltpu.sync_copy(xyzt_hbm.at[b], xyzv)
    inv32 = jnp.float32(1.0 / KNEIGHBORS)

    def body(g, carry):
        gsq_t, m2_t = carry
        pltpu.async_copy(pointsf.at[idxbuf.at[g]], pbuf, sem).wait()
        gsq = jnp.zeros((16,), jnp.float32)
        m2 = jnp.zeros((16,), jnp.float32)
        for j in range(16):
            acc = pbuf[0, pl.ds(16 * j, 16)]
            gsq = gsq + acc * acc
            for k in range(1, KNEIGHBORS):
                v = pbuf[k, pl.ds(16 * j, 16)]
                acc = acc + v
                gsq = gsq + v * v
            sumsb[pl.ds(g * PW + 16 * j, 16)] = acc
            m = acc * inv32
            m2 = m2 + m * m
        n0 = idxbuf[g, pl.ds(0, 16)] - bN
        n1 = idxbuf[g, pl.ds(16, 16)] - bN
        li = _iota16()
        xs = jnp.zeros((16,), jnp.float32)
        for c in range(NDIM):
            x0 = plsc.load_gather(xyzv, [n0 + c * NPTS])
            x1 = plsc.load_gather(xyzv, [n1 + c * NPTS])
            gsq = gsq + x0 * x0
            gsq = gsq + x1 * x1
            sc = jnp.sum(x0 + x1)
            xs = jnp.where(li == c, sc, xs)
        sumsb[pl.ds(g * PW + 256, 16)] = xs
        mx = xs * inv32
        m2 = m2 + mx * mx
        return (gsq_t + gsq, m2_t + m2)

    gsq_t, m2_t = lax.fori_loop(0, GPT, body,
                                (jnp.zeros((16,), jnp.float32),
                                 jnp.zeros((16,), jnp.float32)))
    pltpu.sync_copy(sumsb, sums_hbm.at[b, pl.ds(g0 * PW, GPT * PW)])
    s1 = jnp.sum(gsq_t)
    s2 = jnp.sum(m2_t)
    li = _iota16()
    pv = jnp.where(li == 0, s1, jnp.where(li == 1, s2, jnp.float32(0.0)))
    pvec[...] = pv
    pltpu.sync_copy(pvec, part_hbm.at[w])


def _sc_stats(pointsf, xyzt, gidx):
    k = pl.kernel(
        _sc_stats_body,
        out_type=(
            jax.ShapeDtypeStruct((NBATCH, GROUPS * PW), jnp.float32),
            jax.ShapeDtypeStruct((NW, 16), jnp.float32),
        ),
        mesh=_MESH(),
        compiler_params=pltpu.CompilerParams(use_tc_tiling_on_sc=False, needs_layout_passes=False),
        scratch_types=[
            pltpu.VMEM((GPT, KNEIGHBORS + 1), jnp.int32),
            pltpu.VMEM((NDIM * NPTS,), jnp.float32),
            pltpu.VMEM((GPT * PW,), jnp.float32),
            pltpu.VMEM((KNEIGHBORS + 1, CHANNEL), jnp.float32),
            pltpu.VMEM((16,), jnp.float32),
            pltpu.SemaphoreType.DMA,
        ],
    )
    return k(pointsf, xyzt, gidx)


def _sc_write_body(pointsf, xyzt_hbm, gidx_hbm, sums_hbm, a_hbm, b_hbm,
                   out_hbm,
                   idxbuf, xyzv, sumsb, avec, bvec, pbuf, asm, sem):
    w = lax.axis_index("s") * NC + lax.axis_index("c")
    b = w // TPB
    g0 = (w % TPB) * GPT
    bN = b * NPTS
    pltpu.sync_copy(gidx_hbm.at[b, pl.ds(g0, GPT)], idxbuf)
    pltpu.sync_copy(xyzt_hbm.at[b], xyzv)
    pltpu.sync_copy(sums_hbm.at[b, pl.ds(g0 * PW, GPT * PW)], sumsb)
    pltpu.sync_copy(a_hbm.at[b], avec)
    pltpu.sync_copy(b_hbm.at[b], bvec)
    inv32 = jnp.float32(1.0 / KNEIGHBORS)
    li = _iota16()
    ac = [plsc.load_gather(avec, [_splat_i32(256 + c)]) for c in range(NDIM)]
    bc = [plsc.load_gather(bvec, [_splat_i32(256 + c)]) for c in range(NDIM)]
    off0 = [li * OUTW + (CHANNEL + c) for c in range(NDIM)]
    off1 = [o + 16 * OUTW for o in off0]

    def body(g, carry):
        pltpu.async_copy(pointsf.at[idxbuf.at[g]], pbuf, sem).wait()
        for j in range(16):
            a = avec[pl.ds(16 * j, 16)]
            bb = bvec[pl.ds(16 * j, 16)]
            srow = sumsb[pl.ds(g * PW + 16 * j, 16)]
            sh = bb - (srow * inv32) * a
            for k in range(KNEIGHBORS):
                v = pbuf[k, pl.ds(16 * j, 16)]
                asm[pl.ds(k * OUTW + 16 * j, 16)] = v * a + sh
            rv = pbuf[KNEIGHBORS, pl.ds(16 * j, 16)]
            for k in range(KNEIGHBORS):
                asm[pl.ds(k * OUTW + CTOT + 16 * j, 16)] = rv
        n0 = idxbuf[g, pl.ds(0, 16)] - bN
        n1 = idxbuf[g, pl.ds(16, 16)] - bN
        for c in range(NDIM):
            x0 = plsc.load_gather(xyzv, [n0 + c * NPTS])
            x1 = plsc.load_gather(xyzv, [n1 + c * NPTS])
            mc = plsc.load_gather(sumsb, [_splat_i32(0) + (g * PW + 256 + c)])
            sh = bc[c] - (mc * inv32) * ac[c]
            plsc.store_scatter(asm, [off0[c]], x0 * ac[c] + sh)
            plsc.store_scatter(asm, [off1[c]], x1 * ac[c] + sh)
        gid = b * GROUPS + g0 + g
        pltpu.sync_copy(asm, out_hbm.at[gid])
        return carry

    lax.fori_loop(0, GPT, body, 0)


def _sc_write(pointsf, xyzt, gidx, sums, avals, bvals):
    k = pl.kernel(
        _sc_write_body,
        out_type=jax.ShapeDtypeStruct(
            (NBATCH * GROUPS, KNEIGHBORS * OUTW), jnp.float32),
        mesh=_MESH(),
        compiler_params=pltpu.CompilerParams(use_tc_tiling_on_sc=False, needs_layout_passes=False),
        scratch_types=[
            pltpu.VMEM((GPT, KNEIGHBORS + 1), jnp.int32),
            pltpu.VMEM((NDIM * NPTS,), jnp.float32),
            pltpu.VMEM((GPT * PW,), jnp.float32),
            pltpu.VMEM((PW,), jnp.float32),
            pltpu.VMEM((PW,), jnp.float32),
            pltpu.VMEM((KNEIGHBORS + 1, CHANNEL), jnp.float32),
            pltpu.VMEM((KNEIGHBORS * OUTW,), jnp.float32),
            pltpu.SemaphoreType.DMA,
        ],
    )
    return k(pointsf, xyzt, gidx, sums, avals, bvals)


def kernel(xyz, points, affine_alpha, affine_beta):
    B, N, C = xyz.shape
    S = GROUPS
    K = KNEIGHBORS
    fps_idx, new_xyz = _fps_pallas(xyz)
    dist = -2.0 * jnp.matmul(new_xyz, jnp.transpose(xyz, (0, 2, 1)))
    dist = dist + jnp.sum(new_xyz ** 2, axis=-1)[:, :, None]
    dist = dist + jnp.sum(xyz ** 2, axis=-1)[:, None, :]
    idx = _topk_pallas(dist)

    boff = (jnp.arange(B, dtype=jnp.int32) * N)
    gidx = jnp.concatenate(
        [idx + boff[:, None, None], (fps_idx + boff[:, None])[:, :, None]],
        axis=2).astype(jnp.int32)
    pointsf = points.reshape(B * N, CHANNEL)
    xyzt = jnp.transpose(xyz, (0, 2, 1)).reshape(B, NDIM * N)

    sums, parts = _sc_stats(pointsf, xyzt, gidx)
    pr = parts.reshape(NBATCH, TPB, 16).sum(axis=1)
    ss = pr[:, 0] - K * pr[:, 1]
    M = S * K * CTOT
    std = jnp.sqrt(ss / (M - 1))
    inv = 1.0 / (std + 1e-05)
    af = affine_alpha.reshape(CTOT)
    bf = affine_beta.reshape(CTOT)
    avals = jnp.zeros((NBATCH, PW), jnp.float32).at[:, :CTOT].set(
        inv[:, None] * af[None, :])
    bvals = jnp.zeros((NBATCH, PW), jnp.float32).at[:, :CTOT].set(
        jnp.broadcast_to(bf[None, :], (NBATCH, CTOT)))

    out = _sc_write(pointsf, xyzt, gidx, sums, avals, bvals)
    new_points_out = out.reshape(B, S, K, OUTW)
    return (new_xyz, new_points_out)
